# baseline (speedup 1.0000x reference)
import functools

import jax
import jax.numpy as jnp
from jax import lax
from jax.experimental import pallas as pl
from jax.experimental.pallas import tpu as pltpu
from jax.experimental.pallas import tpu_sc as plsc

N = 100000
D = 128
G = 64
HID = 256
NC, NS = 2, 16
NW = NC * NS

N_SC = 52000
ROWS_W = N_SC // NW
BLK = 125
NBLK = ROWS_W // BLK
BFETCH = 1664
BBUF = 1680
UNROLL = 5
NBUF = 4

R_TC = 800
NB_TC = (N - N_SC) // R_TC
TC_BLK0 = N_SC // R_TC


def _sc_segment_max(x_hbm, b_hbm, out_hbm, bbuf, xbufs, acc, sems):
    wid = lax.axis_index("s") * NC + lax.axis_index("c")
    base = wid * ROWS_W
    abase = (base // 16) * 16
    off = base - abase
    pltpu.sync_copy(b_hbm.at[pl.ds(abase, BFETCH)],
                    bbuf.at[pl.ds(0, BFETCH)])

    neg = jnp.full((16,), -jnp.inf, jnp.float32)
    NJ = D // 16

    def initg(g, _):
        for j in range(NJ):
            acc[g, pl.ds(j * 16, 16)] = neg
        return 0

    lax.fori_loop(0, G, initg, 0)

    def start_copy(blk, buf, sem):
        pltpu.make_async_copy(
            x_hbm.at[pl.ds(base + blk * BLK, BLK), :], buf, sem).start()

    def wait_copy(blk, buf, sem):
        pltpu.make_async_copy(
            x_hbm.at[pl.ds(base + blk * BLK, BLK), :], buf, sem).wait()

    def compute_block(buf, blk, carry):
        sprev = carry[0]
        bidx = off + blk * BLK
        s_first = bbuf[pl.ds(bidx, 16)][0]
        s_last = bbuf[pl.ds(bidx + BLK - 1, 16)][0]
        samev = jnp.full((16,), s_first, jnp.int32) == jnp.full((16,), sprev, jnp.int32)

        def fast(c):
            regs = tuple(jnp.maximum(jnp.where(samev, c[1 + j], neg),
                                     buf[0, pl.ds(j * 16, 16)])
                         for j in range(NJ))

            def step(t, r):
                out = list(r)
                for u in range(UNROLL):
                    i = t * UNROLL + u
                    for j in range(NJ):
                        out[j] = jnp.maximum(out[j], buf[i, pl.ds(j * 16, 16)])
                return tuple(out)

            regs = lax.fori_loop(1, BLK // UNROLL, step, regs)
            for i in range(1, UNROLL):
                regs = tuple(jnp.maximum(regs[j], buf[i, pl.ds(j * 16, 16)])
                             for j in range(NJ))
            for j in range(NJ):
                acc[s_first, pl.ds(j * 16, 16)] = regs[j]
            return (s_first, *regs)

        def slow(c):
            def row(i, cc):
                sp = cc[0]
                s = bbuf[pl.ds(bidx + i, 16)][0]
                sm = jnp.full((16,), s, jnp.int32) == jnp.full((16,), sp, jnp.int32)
                new = []
                for j in range(NJ):
                    rj = buf[i, pl.ds(j * 16, 16)]
                    aj = jnp.maximum(jnp.where(sm, cc[1 + j], neg), rj)
                    acc[s, pl.ds(j * 16, 16)] = aj
                    new.append(aj)
                return (s, *new)

            return lax.fori_loop(0, BLK, row, c)

        return lax.cond(s_first == s_last, fast, slow, carry)

    carry = (jnp.int32(-1),) + tuple(neg for _ in range(NJ))
    for u in range(NBUF):
        start_copy(u, xbufs[u], sems[u])

    NFULL = (NBLK - NBUF) // NBUF
    def turn(k, c):
        for u in range(NBUF):
            b = k * NBUF + u
            wait_copy(b, xbufs[u], sems[u])
            c = compute_block(xbufs[u], b, c)
            start_copy(b + NBUF, xbufs[u], sems[u])
        return c

    carry = lax.fori_loop(0, NFULL, turn, carry)
    for b in range(NFULL * NBUF, NBLK):
        u = b % NBUF
        wait_copy(b, xbufs[u], sems[u])
        carry = compute_block(xbufs[u], b, carry)
        if b + NBUF < NBLK:
            start_copy(b + NBUF, xbufs[u], sems[u])
    pltpu.sync_copy(acc, out_hbm.at[wid])


@functools.partial(
    pl.kernel,
    out_type=jax.ShapeDtypeStruct((NW, G, D), jnp.float32),
    mesh=plsc.VectorSubcoreMesh(core_axis_name="c", subcore_axis_name="s"),
    compiler_params=pltpu.CompilerParams(use_tc_tiling_on_sc=False,
                                         needs_layout_passes=False),
    scratch_types=(
        [pltpu.VMEM((BBUF,), jnp.int32)]
        + [pltpu.VMEM((BLK, D), jnp.float32) for _ in range(NBUF)]
        + [pltpu.VMEM((G, D), jnp.float32)]
        + [pltpu.SemaphoreType.DMA for _ in range(NBUF)]
    ),
)
def _sc_call(x_hbm, b_hbm, out_hbm, bbuf, *rest):
    xbufs = rest[:NBUF]
    acc = rest[NBUF]
    sems = rest[NBUF + 1:]
    _sc_segment_max(x_hbm, b_hbm, out_hbm, bbuf, xbufs, acc, sems)


def _tc_segmax_body(x_ref, b_ref, out_ref):
    pid = pl.program_id(0)

    @pl.when(pid == 0)
    def _():
        out_ref[...] = jnp.full((G, D), -jnp.inf, jnp.float32)

    bb = b_ref[0]
    s_lo = jnp.min(bb)
    s_hi = jnp.max(bb)
    x = x_ref[...]

    def upd(s, m):
        cur = out_ref[pl.ds(s, 1), :]
        out_ref[pl.ds(s, 1), :] = jnp.maximum(cur, m[None, :])

    @pl.when(s_lo == s_hi)
    def _():
        upd(s_lo, jnp.max(x, axis=0))

    @pl.when(s_lo != s_hi)
    def _():
        rio = lax.broadcasted_iota(jnp.int32, (R_TC, D), 0)

        def seg(g, _):
            lt = jnp.sum((bb < g).astype(jnp.int32))
            le = jnp.sum((bb <= g).astype(jnp.int32))
            xm = jnp.where((rio >= lt) & (rio < le), x, -jnp.inf)
            upd(g, jnp.max(xm, axis=0))
            return 0

        lax.fori_loop(s_lo, s_hi + 1, seg, 0)


def _tc_segmax(x, batch_blocks):
    return pl.pallas_call(
        _tc_segmax_body,
        grid=(NB_TC,),
        in_specs=[
            pl.BlockSpec((R_TC, D), lambda b: (TC_BLK0 + b, 0)),
            pl.BlockSpec((1, 1, R_TC), lambda b: (TC_BLK0 + b, 0, 0)),
        ],
        out_specs=pl.BlockSpec((G, D), lambda b: (0, 0)),
        out_shape=jax.ShapeDtypeStruct((G, D), jnp.float32),
    )(x, batch_blocks)


def _mlp_body(part_ref, tc_ref, w1_ref, b1_ref, w2_ref, b2_ref, w3_ref,
              b3_ref, out_ref):
    p = part_ref[...]
    pm = jnp.maximum(jnp.max(p, axis=0), tc_ref[...])
    emb = jnp.where(pm != -jnp.inf, pm, 0.0)
    dn = (((1,), (1,)), ((), ()))
    h = lax.dot_general(emb, w1_ref[...], dn,
                        preferred_element_type=jnp.float32) + b1_ref[...]
    h = jnp.maximum(h, 0.0)
    h = lax.dot_general(h, w2_ref[...], dn,
                        preferred_element_type=jnp.float32) + b2_ref[...]
    h = jnp.maximum(h, 0.0)
    out_ref[...] = lax.dot_general(h, w3_ref[...], dn,
                                   preferred_element_type=jnp.float32) + b3_ref[...]


def kernel(final_node_embeddings, batch_vector, W1, b1, W2, b2, W3, b3):
    batch_i32 = batch_vector.astype(jnp.int32)
    partials = _sc_call(final_node_embeddings, batch_i32)
    batch_blocks = batch_i32.reshape(N // R_TC, 1, R_TC)
    tc_part = _tc_segmax(final_node_embeddings, batch_blocks)
    w3p = jnp.zeros((D, HID), jnp.float32).at[:2, :].set(W3)
    b3p = jnp.zeros((1, D), jnp.float32).at[0, :2].set(b3)
    out = pl.pallas_call(
        _mlp_body,
        out_shape=jax.ShapeDtypeStruct((G, D), jnp.float32),
    )(partials, tc_part, W1, b1[None, :], W2, b2[None, :], w3p, b3p)
    return out[:, :2]


# speedup vs baseline: 1.0079x; 1.0079x over previous
import functools

import jax
import jax.numpy as jnp
from jax import lax
from jax.experimental import pallas as pl
from jax.experimental.pallas import tpu as pltpu
from jax.experimental.pallas import tpu_sc as plsc

N = 100000
D = 128
G = 64
HID = 256
NC, NS = 2, 16
NW = NC * NS

N_SC = 52000
ROWS_W = N_SC // NW
BLK = 125
NBLK = ROWS_W // BLK
BFETCH = 1664
BBUF = 1680
UNROLL = 5
NBUF = 4

R_TC = 800
NB_TC = (N - N_SC) // R_TC
TC_BLK0 = N_SC // R_TC


def _sc_segment_max(x_hbm, b_hbm, out_hbm, bbuf, xbufs, acc, sems):
    wid = lax.axis_index("s") * NC + lax.axis_index("c")
    base = wid * ROWS_W
    abase = (base // 16) * 16
    off = base - abase
    pltpu.sync_copy(b_hbm.at[pl.ds(abase, BFETCH)],
                    bbuf.at[pl.ds(0, BFETCH)])

    neg = jnp.full((16,), -jnp.inf, jnp.float32)
    NJ = D // 16

    def initg(g, _):
        for j in range(NJ):
            acc[g, pl.ds(j * 16, 16)] = neg
        return 0

    lax.fori_loop(0, G, initg, 0)

    def start_copy(blk, buf, sem):
        pltpu.make_async_copy(
            x_hbm.at[pl.ds(base + blk * BLK, BLK), :], buf, sem).start()

    def wait_copy(blk, buf, sem):
        pltpu.make_async_copy(
            x_hbm.at[pl.ds(base + blk * BLK, BLK), :], buf, sem).wait()

    def compute_block(buf, blk, carry):
        sprev = carry[0]
        bidx = off + blk * BLK
        s_first = bbuf[pl.ds(bidx, 16)][0]
        s_last = bbuf[pl.ds(bidx + BLK - 1, 16)][0]
        samev = jnp.full((16,), s_first, jnp.int32) == jnp.full((16,), sprev, jnp.int32)

        def fast(c):
            regs = tuple(jnp.maximum(jnp.where(samev, c[1 + j], neg),
                                     buf[0, pl.ds(j * 16, 16)])
                         for j in range(NJ))

            def step(t, r):
                out = list(r)
                for u in range(UNROLL):
                    i = t * UNROLL + u
                    for j in range(NJ):
                        out[j] = jnp.maximum(out[j], buf[i, pl.ds(j * 16, 16)])
                return tuple(out)

            regs = lax.fori_loop(1, BLK // UNROLL, step, regs)
            for i in range(1, UNROLL):
                regs = tuple(jnp.maximum(regs[j], buf[i, pl.ds(j * 16, 16)])
                             for j in range(NJ))
            for j in range(NJ):
                acc[s_first, pl.ds(j * 16, 16)] = regs[j]
            return (s_first, *regs)

        def slow(c):
            def row(i, cc):
                sp = cc[0]
                s = bbuf[pl.ds(bidx + i, 16)][0]
                sm = jnp.full((16,), s, jnp.int32) == jnp.full((16,), sp, jnp.int32)
                new = []
                for j in range(NJ):
                    rj = buf[i, pl.ds(j * 16, 16)]
                    aj = jnp.maximum(jnp.where(sm, cc[1 + j], neg), rj)
                    acc[s, pl.ds(j * 16, 16)] = aj
                    new.append(aj)
                return (s, *new)

            return lax.fori_loop(0, BLK, row, c)

        return lax.cond(s_first == s_last, fast, slow, carry)

    carry = (jnp.int32(-1),) + tuple(neg for _ in range(NJ))
    for u in range(NBUF):
        start_copy(u, xbufs[u], sems[u])

    NFULL = (NBLK - NBUF) // NBUF
    def turn(k, c):
        for u in range(NBUF):
            b = k * NBUF + u
            wait_copy(b, xbufs[u], sems[u])
            c = compute_block(xbufs[u], b, c)
            start_copy(b + NBUF, xbufs[u], sems[u])
        return c

    carry = lax.fori_loop(0, NFULL, turn, carry)
    for b in range(NFULL * NBUF, NBLK):
        u = b % NBUF
        wait_copy(b, xbufs[u], sems[u])
        carry = compute_block(xbufs[u], b, carry)
        if b + NBUF < NBLK:
            start_copy(b + NBUF, xbufs[u], sems[u])
    pltpu.sync_copy(acc, out_hbm.at[wid])


@functools.partial(
    pl.kernel,
    out_type=jax.ShapeDtypeStruct((NW, G, D), jnp.float32),
    mesh=plsc.VectorSubcoreMesh(core_axis_name="c", subcore_axis_name="s"),
    compiler_params=pltpu.CompilerParams(use_tc_tiling_on_sc=False,
                                         needs_layout_passes=False),
    scratch_types=(
        [pltpu.VMEM((BBUF,), jnp.int32)]
        + [pltpu.VMEM((BLK, D), jnp.float32) for _ in range(NBUF)]
        + [pltpu.VMEM((G, D), jnp.float32)]
        + [pltpu.SemaphoreType.DMA for _ in range(NBUF)]
    ),
)
def _sc_call(x_hbm, b_hbm, out_hbm, bbuf, *rest):
    xbufs = rest[:NBUF]
    acc = rest[NBUF]
    sems = rest[NBUF + 1:]
    _sc_segment_max(x_hbm, b_hbm, out_hbm, bbuf, xbufs, acc, sems)


def _tc_segmax_body(x_ref, b_ref, out_ref):
    pid = pl.program_id(0)

    @pl.when(pid == 0)
    def _():
        out_ref[...] = jnp.full((G, D), -jnp.inf, jnp.float32)

    bb = b_ref[pl.ds(pid, 1), :]
    s_lo = jnp.min(bb)
    s_hi = jnp.max(bb)
    x = x_ref[...]

    def upd(s, m):
        cur = out_ref[pl.ds(s, 1), :]
        out_ref[pl.ds(s, 1), :] = jnp.maximum(cur, m[None, :])

    @pl.when(s_lo == s_hi)
    def _():
        upd(s_lo, jnp.max(x, axis=0))

    @pl.when(s_lo != s_hi)
    def _():
        rio = lax.broadcasted_iota(jnp.int32, (R_TC, D), 0)

        def seg(g, _):
            lt = jnp.sum((bb < g).astype(jnp.int32))
            le = jnp.sum((bb <= g).astype(jnp.int32))
            xm = jnp.where((rio >= lt) & (rio < le), x, -jnp.inf)
            upd(g, jnp.max(xm, axis=0))
            return 0

        lax.fori_loop(s_lo, s_hi + 1, seg, 0)


def _tc_segmax(x, batch_blocks):
    return pl.pallas_call(
        _tc_segmax_body,
        grid=(NB_TC,),
        in_specs=[
            pl.BlockSpec((R_TC, D), lambda b: (TC_BLK0 + b, 0)),
            pl.BlockSpec((NB_TC, R_TC), lambda b: (0, 0)),
        ],
        out_specs=pl.BlockSpec((G, D), lambda b: (0, 0)),
        out_shape=jax.ShapeDtypeStruct((G, D), jnp.float32),
    )(x, batch_blocks)


def _mlp_body(part_ref, tc_ref, w1_ref, b1_ref, w2_ref, b2_ref, w3_ref,
              b3_ref, out_ref):
    p = part_ref[...]
    pm = jnp.maximum(jnp.max(p, axis=0), tc_ref[...])
    emb = jnp.where(pm != -jnp.inf, pm, 0.0)
    dn = (((1,), (1,)), ((), ()))
    h = lax.dot_general(emb, w1_ref[...], dn,
                        preferred_element_type=jnp.float32) + b1_ref[...]
    h = jnp.maximum(h, 0.0)
    h = lax.dot_general(h, w2_ref[...], dn,
                        preferred_element_type=jnp.float32) + b2_ref[...]
    h = jnp.maximum(h, 0.0)
    out_ref[...] = lax.dot_general(h, w3_ref[...], dn,
                                   preferred_element_type=jnp.float32) + b3_ref[...]


def kernel(final_node_embeddings, batch_vector, W1, b1, W2, b2, W3, b3):
    batch_i32 = batch_vector.astype(jnp.int32)
    partials = _sc_call(final_node_embeddings, batch_i32)
    batch_blocks = batch_i32[N_SC:].reshape(NB_TC, R_TC)
    tc_part = _tc_segmax(final_node_embeddings, batch_blocks)
    w3p = jnp.zeros((D, HID), jnp.float32).at[:2, :].set(W3)
    b3p = jnp.zeros((1, D), jnp.float32).at[0, :2].set(b3)
    out = pl.pallas_call(
        _mlp_body,
        out_shape=jax.ShapeDtypeStruct((G, D), jnp.float32),
    )(partials, tc_part, W1, b1[None, :], W2, b2[None, :], w3p, b3p)
    return out[:, :2]


# speedup vs baseline: 1.3177x; 1.3073x over previous
import functools

import jax
import jax.numpy as jnp
from jax import lax
from jax.experimental import pallas as pl
from jax.experimental.pallas import tpu as pltpu
from jax.experimental.pallas import tpu_sc as plsc

N = 100000
D = 128
G = 64
HID = 256
NC, NS = 2, 16
NW = NC * NS

N_SC = 52000
ROWS_W = N_SC // NW
BLK = 125
NBLK = ROWS_W // BLK
BFETCH = 1664
BBUF = 1680
UNROLL = 5
NBUF = 4

R_TC = 2000
NB_TC = (N - N_SC) // R_TC
TC_BLK0 = N_SC // R_TC


def _sc_segment_max(x_hbm, b_hbm, out_hbm, bbuf, xbufs, acc, sems):
    wid = lax.axis_index("s") * NC + lax.axis_index("c")
    base = wid * ROWS_W
    abase = (base // 16) * 16
    off = base - abase
    pltpu.sync_copy(b_hbm.at[pl.ds(abase, BFETCH)],
                    bbuf.at[pl.ds(0, BFETCH)])

    neg = jnp.full((16,), -jnp.inf, jnp.float32)
    NJ = D // 16

    def initg(g, _):
        for j in range(NJ):
            acc[g, pl.ds(j * 16, 16)] = neg
        return 0

    lax.fori_loop(0, G, initg, 0)

    def start_copy(blk, buf, sem):
        pltpu.make_async_copy(
            x_hbm.at[pl.ds(base + blk * BLK, BLK), :], buf, sem).start()

    def wait_copy(blk, buf, sem):
        pltpu.make_async_copy(
            x_hbm.at[pl.ds(base + blk * BLK, BLK), :], buf, sem).wait()

    def compute_block(buf, blk, carry):
        sprev = carry[0]
        bidx = off + blk * BLK
        s_first = bbuf[pl.ds(bidx, 16)][0]
        s_last = bbuf[pl.ds(bidx + BLK - 1, 16)][0]
        samev = jnp.full((16,), s_first, jnp.int32) == jnp.full((16,), sprev, jnp.int32)

        def fast(c):
            regs = tuple(jnp.maximum(jnp.where(samev, c[1 + j], neg),
                                     buf[0, pl.ds(j * 16, 16)])
                         for j in range(NJ))

            def step(t, r):
                out = list(r)
                for u in range(UNROLL):
                    i = t * UNROLL + u
                    for j in range(NJ):
                        out[j] = jnp.maximum(out[j], buf[i, pl.ds(j * 16, 16)])
                return tuple(out)

            regs = lax.fori_loop(1, BLK // UNROLL, step, regs)
            for i in range(1, UNROLL):
                regs = tuple(jnp.maximum(regs[j], buf[i, pl.ds(j * 16, 16)])
                             for j in range(NJ))
            for j in range(NJ):
                acc[s_first, pl.ds(j * 16, 16)] = regs[j]
            return (s_first, *regs)

        def slow(c):
            def row(i, cc):
                sp = cc[0]
                s = bbuf[pl.ds(bidx + i, 16)][0]
                sm = jnp.full((16,), s, jnp.int32) == jnp.full((16,), sp, jnp.int32)
                new = []
                for j in range(NJ):
                    rj = buf[i, pl.ds(j * 16, 16)]
                    aj = jnp.maximum(jnp.where(sm, cc[1 + j], neg), rj)
                    acc[s, pl.ds(j * 16, 16)] = aj
                    new.append(aj)
                return (s, *new)

            return lax.fori_loop(0, BLK, row, c)

        return lax.cond(s_first == s_last, fast, slow, carry)

    carry = (jnp.int32(-1),) + tuple(neg for _ in range(NJ))
    for u in range(NBUF):
        start_copy(u, xbufs[u], sems[u])

    NFULL = (NBLK - NBUF) // NBUF
    def turn(k, c):
        for u in range(NBUF):
            b = k * NBUF + u
            wait_copy(b, xbufs[u], sems[u])
            c = compute_block(xbufs[u], b, c)
            start_copy(b + NBUF, xbufs[u], sems[u])
        return c

    carry = lax.fori_loop(0, NFULL, turn, carry)
    for b in range(NFULL * NBUF, NBLK):
        u = b % NBUF
        wait_copy(b, xbufs[u], sems[u])
        carry = compute_block(xbufs[u], b, carry)
        if b + NBUF < NBLK:
            start_copy(b + NBUF, xbufs[u], sems[u])
    pltpu.sync_copy(acc, out_hbm.at[wid])


@functools.partial(
    pl.kernel,
    out_type=jax.ShapeDtypeStruct((NW, G, D), jnp.float32),
    mesh=plsc.VectorSubcoreMesh(core_axis_name="c", subcore_axis_name="s"),
    compiler_params=pltpu.CompilerParams(use_tc_tiling_on_sc=False,
                                         needs_layout_passes=False),
    scratch_types=(
        [pltpu.VMEM((BBUF,), jnp.int32)]
        + [pltpu.VMEM((BLK, D), jnp.float32) for _ in range(NBUF)]
        + [pltpu.VMEM((G, D), jnp.float32)]
        + [pltpu.SemaphoreType.DMA for _ in range(NBUF)]
    ),
)
def _sc_call(x_hbm, b_hbm, out_hbm, bbuf, *rest):
    xbufs = rest[:NBUF]
    acc = rest[NBUF]
    sems = rest[NBUF + 1:]
    _sc_segment_max(x_hbm, b_hbm, out_hbm, bbuf, xbufs, acc, sems)


def _tc_segmax_body(x_ref, b_ref, out_ref):
    pid = pl.program_id(0)

    @pl.when(pid == 0)
    def _():
        out_ref[...] = jnp.full((G, D), -jnp.inf, jnp.float32)

    bb = b_ref[pl.ds(pid, 1), :]
    s_lo = jnp.min(bb)
    s_hi = jnp.max(bb)
    x = x_ref[...]

    def upd(s, m):
        cur = out_ref[pl.ds(s, 1), :]
        out_ref[pl.ds(s, 1), :] = jnp.maximum(cur, m[None, :])

    @pl.when(s_lo == s_hi)
    def _():
        upd(s_lo, jnp.max(x, axis=0))

    @pl.when(s_lo != s_hi)
    def _():
        rio = lax.broadcasted_iota(jnp.int32, (R_TC, D), 0)

        def seg(g, _):
            lt = jnp.sum((bb < g).astype(jnp.int32))
            le = jnp.sum((bb <= g).astype(jnp.int32))
            xm = jnp.where((rio >= lt) & (rio < le), x, -jnp.inf)
            upd(g, jnp.max(xm, axis=0))
            return 0

        lax.fori_loop(s_lo, s_hi + 1, seg, 0)


def _tc_segmax(x, batch_blocks):
    return pl.pallas_call(
        _tc_segmax_body,
        grid=(NB_TC,),
        in_specs=[
            pl.BlockSpec((R_TC, D), lambda b: (TC_BLK0 + b, 0)),
            pl.BlockSpec((NB_TC, R_TC), lambda b: (0, 0)),
        ],
        out_specs=pl.BlockSpec((G, D), lambda b: (0, 0)),
        out_shape=jax.ShapeDtypeStruct((G, D), jnp.float32),
    )(x, batch_blocks)


def _mlp_body(part_ref, tc_ref, w1_ref, b1_ref, w2_ref, b2_ref, w3_ref,
              b3_ref, out_ref):
    p = part_ref[...]
    pm = jnp.maximum(jnp.max(p, axis=0), tc_ref[...])
    emb = jnp.where(pm != -jnp.inf, pm, 0.0)
    dn = (((1,), (1,)), ((), ()))
    h = lax.dot_general(emb, w1_ref[...], dn,
                        preferred_element_type=jnp.float32) + b1_ref[...]
    h = jnp.maximum(h, 0.0)
    h = lax.dot_general(h, w2_ref[...], dn,
                        preferred_element_type=jnp.float32) + b2_ref[...]
    h = jnp.maximum(h, 0.0)
    out_ref[...] = lax.dot_general(h, w3_ref[...], dn,
                                   preferred_element_type=jnp.float32) + b3_ref[...]


def kernel(final_node_embeddings, batch_vector, W1, b1, W2, b2, W3, b3):
    batch_i32 = batch_vector.astype(jnp.int32)
    partials = _sc_call(final_node_embeddings, batch_i32)
    batch_blocks = batch_i32[N_SC:].reshape(NB_TC, R_TC)
    tc_part = _tc_segmax(final_node_embeddings, batch_blocks)
    w3p = jnp.zeros((D, HID), jnp.float32).at[:2, :].set(W3)
    b3p = jnp.zeros((1, D), jnp.float32).at[0, :2].set(b3)
    out = pl.pallas_call(
        _mlp_body,
        out_shape=jax.ShapeDtypeStruct((G, D), jnp.float32),
    )(partials, tc_part, W1, b1[None, :], W2, b2[None, :], w3p, b3p)
    return out[:, :2]


# speedup vs baseline: 1.3319x; 1.0108x over previous
import functools

import jax
import jax.numpy as jnp
from jax import lax
from jax.experimental import pallas as pl
from jax.experimental.pallas import tpu as pltpu
from jax.experimental.pallas import tpu_sc as plsc

N = 100000
D = 128
G = 64
HID = 256
NC, NS = 2, 16
NW = NC * NS

N_SC = 52000
ROWS_W = N_SC // NW
BLK = 125
NBLK = ROWS_W // BLK
BFETCH = 1664
BBUF = 1680
UNROLL = 5
NBUF = 4

R_TC = 4000
NB_TC = (N - N_SC) // R_TC
TC_BLK0 = N_SC // R_TC


def _sc_segment_max(x_hbm, b_hbm, out_hbm, bbuf, xbufs, acc, sems):
    wid = lax.axis_index("s") * NC + lax.axis_index("c")
    base = wid * ROWS_W
    abase = (base // 16) * 16
    off = base - abase
    pltpu.sync_copy(b_hbm.at[pl.ds(abase, BFETCH)],
                    bbuf.at[pl.ds(0, BFETCH)])

    neg = jnp.full((16,), -jnp.inf, jnp.float32)
    NJ = D // 16

    def initg(g, _):
        for j in range(NJ):
            acc[g, pl.ds(j * 16, 16)] = neg
        return 0

    lax.fori_loop(0, G, initg, 0)

    def start_copy(blk, buf, sem):
        pltpu.make_async_copy(
            x_hbm.at[pl.ds(base + blk * BLK, BLK), :], buf, sem).start()

    def wait_copy(blk, buf, sem):
        pltpu.make_async_copy(
            x_hbm.at[pl.ds(base + blk * BLK, BLK), :], buf, sem).wait()

    def compute_block(buf, blk, carry):
        sprev = carry[0]
        bidx = off + blk * BLK
        s_first = bbuf[pl.ds(bidx, 16)][0]
        s_last = bbuf[pl.ds(bidx + BLK - 1, 16)][0]
        samev = jnp.full((16,), s_first, jnp.int32) == jnp.full((16,), sprev, jnp.int32)

        def fast(c):
            regs = tuple(jnp.maximum(jnp.where(samev, c[1 + j], neg),
                                     buf[0, pl.ds(j * 16, 16)])
                         for j in range(NJ))

            def step(t, r):
                out = list(r)
                for u in range(UNROLL):
                    i = t * UNROLL + u
                    for j in range(NJ):
                        out[j] = jnp.maximum(out[j], buf[i, pl.ds(j * 16, 16)])
                return tuple(out)

            regs = lax.fori_loop(1, BLK // UNROLL, step, regs)
            for i in range(1, UNROLL):
                regs = tuple(jnp.maximum(regs[j], buf[i, pl.ds(j * 16, 16)])
                             for j in range(NJ))
            for j in range(NJ):
                acc[s_first, pl.ds(j * 16, 16)] = regs[j]
            return (s_first, *regs)

        def slow(c):
            def row(i, cc):
                sp = cc[0]
                s = bbuf[pl.ds(bidx + i, 16)][0]
                sm = jnp.full((16,), s, jnp.int32) == jnp.full((16,), sp, jnp.int32)
                new = []
                for j in range(NJ):
                    rj = buf[i, pl.ds(j * 16, 16)]
                    aj = jnp.maximum(jnp.where(sm, cc[1 + j], neg), rj)
                    acc[s, pl.ds(j * 16, 16)] = aj
                    new.append(aj)
                return (s, *new)

            return lax.fori_loop(0, BLK, row, c)

        return lax.cond(s_first == s_last, fast, slow, carry)

    carry = (jnp.int32(-1),) + tuple(neg for _ in range(NJ))
    for u in range(NBUF):
        start_copy(u, xbufs[u], sems[u])

    NFULL = (NBLK - NBUF) // NBUF
    def turn(k, c):
        for u in range(NBUF):
            b = k * NBUF + u
            wait_copy(b, xbufs[u], sems[u])
            c = compute_block(xbufs[u], b, c)
            start_copy(b + NBUF, xbufs[u], sems[u])
        return c

    carry = lax.fori_loop(0, NFULL, turn, carry)
    for b in range(NFULL * NBUF, NBLK):
        u = b % NBUF
        wait_copy(b, xbufs[u], sems[u])
        carry = compute_block(xbufs[u], b, carry)
        if b + NBUF < NBLK:
            start_copy(b + NBUF, xbufs[u], sems[u])
    pltpu.sync_copy(acc, out_hbm.at[wid])


@functools.partial(
    pl.kernel,
    out_type=jax.ShapeDtypeStruct((NW, G, D), jnp.float32),
    mesh=plsc.VectorSubcoreMesh(core_axis_name="c", subcore_axis_name="s"),
    compiler_params=pltpu.CompilerParams(use_tc_tiling_on_sc=False,
                                         needs_layout_passes=False),
    scratch_types=(
        [pltpu.VMEM((BBUF,), jnp.int32)]
        + [pltpu.VMEM((BLK, D), jnp.float32) for _ in range(NBUF)]
        + [pltpu.VMEM((G, D), jnp.float32)]
        + [pltpu.SemaphoreType.DMA for _ in range(NBUF)]
    ),
)
def _sc_call(x_hbm, b_hbm, out_hbm, bbuf, *rest):
    xbufs = rest[:NBUF]
    acc = rest[NBUF]
    sems = rest[NBUF + 1:]
    _sc_segment_max(x_hbm, b_hbm, out_hbm, bbuf, xbufs, acc, sems)


def _tc_segmax_body(x_ref, b_ref, out_ref):
    pid = pl.program_id(0)

    @pl.when(pid == 0)
    def _():
        out_ref[...] = jnp.full((G, D), -jnp.inf, jnp.float32)

    bb = b_ref[pl.ds(pid, 1), :]
    s_lo = jnp.min(bb)
    s_hi = jnp.max(bb)
    x = x_ref[...]

    def upd(s, m):
        cur = out_ref[pl.ds(s, 1), :]
        out_ref[pl.ds(s, 1), :] = jnp.maximum(cur, m[None, :])

    @pl.when(s_lo == s_hi)
    def _():
        upd(s_lo, jnp.max(x, axis=0))

    @pl.when(s_lo != s_hi)
    def _():
        rio = lax.broadcasted_iota(jnp.int32, (R_TC, D), 0)

        def seg(g, _):
            lt = jnp.sum((bb < g).astype(jnp.int32))
            le = jnp.sum((bb <= g).astype(jnp.int32))
            xm = jnp.where((rio >= lt) & (rio < le), x, -jnp.inf)
            upd(g, jnp.max(xm, axis=0))
            return 0

        lax.fori_loop(s_lo, s_hi + 1, seg, 0)


def _tc_segmax(x, batch_blocks):
    return pl.pallas_call(
        _tc_segmax_body,
        grid=(NB_TC,),
        in_specs=[
            pl.BlockSpec((R_TC, D), lambda b: (TC_BLK0 + b, 0)),
            pl.BlockSpec((NB_TC, R_TC), lambda b: (0, 0)),
        ],
        out_specs=pl.BlockSpec((G, D), lambda b: (0, 0)),
        out_shape=jax.ShapeDtypeStruct((G, D), jnp.float32),
    )(x, batch_blocks)


def _mlp_body(part_ref, tc_ref, w1_ref, b1_ref, w2_ref, b2_ref, w3_ref,
              b3_ref, out_ref):
    p = part_ref[...]
    pm = jnp.maximum(jnp.max(p, axis=0), tc_ref[...])
    emb = jnp.where(pm != -jnp.inf, pm, 0.0)
    dn = (((1,), (1,)), ((), ()))
    h = lax.dot_general(emb, w1_ref[...], dn,
                        preferred_element_type=jnp.float32) + b1_ref[...]
    h = jnp.maximum(h, 0.0)
    h = lax.dot_general(h, w2_ref[...], dn,
                        preferred_element_type=jnp.float32) + b2_ref[...]
    h = jnp.maximum(h, 0.0)
    out_ref[...] = lax.dot_general(h, w3_ref[...], dn,
                                   preferred_element_type=jnp.float32) + b3_ref[...]


def kernel(final_node_embeddings, batch_vector, W1, b1, W2, b2, W3, b3):
    batch_i32 = batch_vector.astype(jnp.int32)
    partials = _sc_call(final_node_embeddings, batch_i32)
    batch_blocks = batch_i32[N_SC:].reshape(NB_TC, R_TC)
    tc_part = _tc_segmax(final_node_embeddings, batch_blocks)
    w3p = jnp.zeros((D, HID), jnp.float32).at[:2, :].set(W3)
    b3p = jnp.zeros((1, D), jnp.float32).at[0, :2].set(b3)
    out = pl.pallas_call(
        _mlp_body,
        out_shape=jax.ShapeDtypeStruct((G, D), jnp.float32),
    )(partials, tc_part, W1, b1[None, :], W2, b2[None, :], w3p, b3p)
    return out[:, :2]


# speedup vs baseline: 1.3346x; 1.0020x over previous
import functools

import jax
import jax.numpy as jnp
from jax import lax
from jax.experimental import pallas as pl
from jax.experimental.pallas import tpu as pltpu
from jax.experimental.pallas import tpu_sc as plsc

N = 100000
D = 128
G = 64
HID = 256
NC, NS = 2, 16
NW = NC * NS

N_SC = 52000
ROWS_W = N_SC // NW
BLK = 125
NBLK = ROWS_W // BLK
BFETCH = 1664
BBUF = 1680
UNROLL = 5
NBUF = 4

R_TC = 4000
NB_TC = (N - N_SC) // R_TC
TC_BLK0 = N_SC // R_TC


def _sc_segment_max(x_hbm, b_hbm, out_hbm, bbuf, xbufs, acc, sems):
    wid = lax.axis_index("s") * NC + lax.axis_index("c")
    base = wid * ROWS_W
    abase = (base // 16) * 16
    off = base - abase
    pltpu.sync_copy(b_hbm.at[pl.ds(abase, BFETCH)],
                    bbuf.at[pl.ds(0, BFETCH)])

    neg = jnp.full((16,), -jnp.inf, jnp.float32)
    NJ = D // 16

    def initg(g, _):
        for j in range(NJ):
            acc[g, pl.ds(j * 16, 16)] = neg
        return 0

    lax.fori_loop(0, G, initg, 0)

    def start_copy(blk, buf, sem):
        pltpu.make_async_copy(
            x_hbm.at[pl.ds(base + blk * BLK, BLK), :], buf, sem).start()

    def wait_copy(blk, buf, sem):
        pltpu.make_async_copy(
            x_hbm.at[pl.ds(base + blk * BLK, BLK), :], buf, sem).wait()

    def compute_block(buf, blk, carry):
        sprev = carry[0]
        bidx = off + blk * BLK
        s_first = bbuf[pl.ds(bidx, 16)][0]
        s_last = bbuf[pl.ds(bidx + BLK - 1, 16)][0]
        samev = jnp.full((16,), s_first, jnp.int32) == jnp.full((16,), sprev, jnp.int32)

        def fast(c):
            regs = tuple(jnp.maximum(jnp.where(samev, c[1 + j], neg),
                                     buf[0, pl.ds(j * 16, 16)])
                         for j in range(NJ))

            def step(t, r):
                out = list(r)
                for u in range(UNROLL):
                    i = t * UNROLL + u
                    for j in range(NJ):
                        out[j] = jnp.maximum(out[j], buf[i, pl.ds(j * 16, 16)])
                return tuple(out)

            regs = lax.fori_loop(1, BLK // UNROLL, step, regs)
            for i in range(1, UNROLL):
                regs = tuple(jnp.maximum(regs[j], buf[i, pl.ds(j * 16, 16)])
                             for j in range(NJ))
            for j in range(NJ):
                acc[s_first, pl.ds(j * 16, 16)] = regs[j]
            return (s_first, *regs)

        def slow(c):
            def row(i, cc):
                sp = cc[0]
                s = bbuf[pl.ds(bidx + i, 16)][0]
                sm = jnp.full((16,), s, jnp.int32) == jnp.full((16,), sp, jnp.int32)
                new = []
                for j in range(NJ):
                    rj = buf[i, pl.ds(j * 16, 16)]
                    aj = jnp.maximum(jnp.where(sm, cc[1 + j], neg), rj)
                    acc[s, pl.ds(j * 16, 16)] = aj
                    new.append(aj)
                return (s, *new)

            return lax.fori_loop(0, BLK, row, c)

        return lax.cond(s_first == s_last, fast, slow, carry)

    carry = (jnp.int32(-1),) + tuple(neg for _ in range(NJ))
    for u in range(NBUF):
        start_copy(u, xbufs[u], sems[u])

    NFULL = (NBLK - NBUF) // NBUF
    def turn(k, c):
        for u in range(NBUF):
            b = k * NBUF + u
            wait_copy(b, xbufs[u], sems[u])
            c = compute_block(xbufs[u], b, c)
            start_copy(b + NBUF, xbufs[u], sems[u])
        return c

    carry = lax.fori_loop(0, NFULL, turn, carry)
    for b in range(NFULL * NBUF, NBLK):
        u = b % NBUF
        wait_copy(b, xbufs[u], sems[u])
        carry = compute_block(xbufs[u], b, carry)
        if b + NBUF < NBLK:
            start_copy(b + NBUF, xbufs[u], sems[u])
    pltpu.sync_copy(acc, out_hbm.at[wid])


@functools.partial(
    pl.kernel,
    out_type=jax.ShapeDtypeStruct((NW, G, D), jnp.float32),
    mesh=plsc.VectorSubcoreMesh(core_axis_name="c", subcore_axis_name="s"),
    compiler_params=pltpu.CompilerParams(use_tc_tiling_on_sc=False,
                                         needs_layout_passes=False),
    scratch_types=(
        [pltpu.VMEM((BBUF,), jnp.int32)]
        + [pltpu.VMEM((BLK, D), jnp.float32) for _ in range(NBUF)]
        + [pltpu.VMEM((G, D), jnp.float32)]
        + [pltpu.SemaphoreType.DMA for _ in range(NBUF)]
    ),
)
def _sc_call(x_hbm, b_hbm, out_hbm, bbuf, *rest):
    xbufs = rest[:NBUF]
    acc = rest[NBUF]
    sems = rest[NBUF + 1:]
    _sc_segment_max(x_hbm, b_hbm, out_hbm, bbuf, xbufs, acc, sems)


def _tc_segmax_body(x_ref, b_ref, out_ref):
    pid = pl.program_id(0)

    @pl.when(pid == 0)
    def _():
        out_ref[...] = jnp.full((G, D), -jnp.inf, jnp.float32)

    bb = b_ref[pl.ds(pid, 1), :]
    s_lo = jnp.min(bb)
    s_hi = jnp.max(bb)
    x = x_ref[...]

    def upd(s, m):
        cur = out_ref[pl.ds(s, 1), :]
        out_ref[pl.ds(s, 1), :] = jnp.maximum(cur, m[None, :])

    @pl.when(s_lo == s_hi)
    def _():
        upd(s_lo, jnp.max(x, axis=0))

    @pl.when(s_lo != s_hi)
    def _():
        rio = lax.broadcasted_iota(jnp.int32, (R_TC, D), 0)

        def seg(g, _):
            lt = jnp.sum((bb < g).astype(jnp.int32))
            le = jnp.sum((bb <= g).astype(jnp.int32))
            xm = jnp.where((rio >= lt) & (rio < le), x, -jnp.inf)
            upd(g, jnp.max(xm, axis=0))
            return 0

        lax.fori_loop(s_lo, s_hi + 1, seg, 0)


def _tc_segmax(x, batch_blocks):
    return pl.pallas_call(
        _tc_segmax_body,
        grid=(NB_TC,),
        in_specs=[
            pl.BlockSpec((R_TC, D), lambda b: (TC_BLK0 + b, 0)),
            pl.BlockSpec((NB_TC, R_TC), lambda b: (0, 0)),
        ],
        out_specs=pl.BlockSpec((G, D), lambda b: (0, 0)),
        out_shape=jax.ShapeDtypeStruct((G, D), jnp.float32),
    )(x, batch_blocks)


def _mlp_body(part_ref, tc_ref, w1_ref, b1_ref, w2_ref, b2_ref, w3_ref,
              b3_ref, out_ref):
    p = part_ref[...]
    pm = jnp.maximum(jnp.max(p, axis=0), tc_ref[...])
    emb = jnp.where(pm != -jnp.inf, pm, 0.0)
    dn = (((1,), (1,)), ((), ()))
    h = lax.dot_general(emb, w1_ref[...], dn,
                        preferred_element_type=jnp.float32) + b1_ref[...]
    h = jnp.maximum(h, 0.0)
    h = lax.dot_general(h, w2_ref[...], dn,
                        preferred_element_type=jnp.float32) + b2_ref[...]
    h = jnp.maximum(h, 0.0)
    out_ref[...] = lax.dot_general(h, w3_ref[...], dn,
                                   preferred_element_type=jnp.float32) + b3_ref[...]


def kernel(final_node_embeddings, batch_vector, W1, b1, W2, b2, W3, b3):
    batch_i32 = batch_vector.astype(jnp.int32)
    batch_blocks = batch_i32[N_SC:].reshape(NB_TC, R_TC)
    partials = _sc_call(final_node_embeddings, batch_i32)
    tc_part = _tc_segmax(final_node_embeddings, batch_blocks)
    w3p = jnp.zeros((D, HID), jnp.float32).at[:2, :].set(W3)
    b3p = jnp.zeros((1, D), jnp.float32).at[0, :2].set(b3)
    out = pl.pallas_call(
        _mlp_body,
        out_shape=jax.ShapeDtypeStruct((G, D), jnp.float32),
    )(partials, tc_part, W1, b1[None, :], W2, b2[None, :], w3p, b3p)
    return out[:, :2]
